# baseline (speedup 1.0000x reference)
import functools

import jax
import jax.numpy as jnp
from jax import lax
from jax.experimental import pallas as pl
from jax.experimental.pallas import tpu as pltpu
from jax.experimental.pallas import tpu_sc as plsc

_EMBED = 128
_BATCH = 16384
_NC = 2
_NS = 16
_NW = _NC * _NS
_CHUNKS = 2
_CB = _BATCH // _CHUNKS
_B_PER_W = _CB // _NW


def _gather_body(user_tab, book_tab, uids, bids, u_out, b_out, idx_v, rows_v, sem):
    wid = lax.axis_index("s") * _NC + lax.axis_index("c")
    base = wid * _B_PER_W
    pltpu.sync_copy(uids.at[pl.ds(base, _B_PER_W)], idx_v)
    pltpu.async_copy(user_tab.at[idx_v], rows_v, sem).wait()
    pltpu.sync_copy(rows_v, u_out.at[pl.ds(base, _B_PER_W)])
    pltpu.sync_copy(bids.at[pl.ds(base, _B_PER_W)], idx_v)
    pltpu.async_copy(book_tab.at[idx_v], rows_v, sem).wait()
    pltpu.sync_copy(rows_v, b_out.at[pl.ds(base, _B_PER_W)])


_sc_gather = functools.partial(
    pl.kernel,
    mesh=plsc.VectorSubcoreMesh(core_axis_name="c", subcore_axis_name="s"),
    out_type=[
        jax.ShapeDtypeStruct((_CB, _EMBED), jnp.float32),
        jax.ShapeDtypeStruct((_CB, _EMBED), jnp.float32),
    ],
    scratch_types=[
        pltpu.VMEM((_B_PER_W,), jnp.int32),
        pltpu.VMEM((_B_PER_W, _EMBED), jnp.float32),
        pltpu.SemaphoreType.DMA,
    ],
)(_gather_body)


_BLK = 2048

_DNUMS = (((0,), (0,)), ((), ()))


def _mlp_body(u_ref, b_ref, w0_ref, b0_ref, w1_ref, b1_ref,
              w2_ref, b2_ref, w3_ref, b3_ref, out_ref, xt_ref):
    f32, bf16 = jnp.float32, jnp.bfloat16
    dot = functools.partial(lax.dot_general, dimension_numbers=_DNUMS,
                            preferred_element_type=f32)
    xt_ref[:_EMBED, :] = u_ref[...].astype(bf16).T
    xt_ref[_EMBED:, :] = b_ref[...].astype(bf16).T
    h = dot(w0_ref[...].astype(bf16), xt_ref[...])
    h = jnp.maximum(h + b0_ref[...], 0.0).astype(bf16)
    h = dot(w1_ref[...].astype(bf16), h)
    h = jnp.maximum(h + b1_ref[...], 0.0).astype(bf16)
    h = dot(w2_ref[...].astype(bf16), h)
    h = jnp.maximum(h + b2_ref[...], 0.0).astype(bf16)
    y = dot(w3_ref[...].astype(bf16), h)
    out_ref[...] = y[0] + b3_ref[0]


def _mlp(u, b, w0, b0, w1, b1, w2, b2, w3, b3):
    grid = _CB // _BLK
    full = lambda shape: pl.BlockSpec(shape, lambda i: (0,) * len(shape))
    return pl.pallas_call(
        _mlp_body,
        grid=(grid,),
        in_specs=[
            pl.BlockSpec((_BLK, _EMBED), lambda i: (i, 0)),
            pl.BlockSpec((_BLK, _EMBED), lambda i: (i, 0)),
            full((2 * _EMBED, 512)),
            full((512, 1)),
            full((512, 256)),
            full((256, 1)),
            full((256, 128)),
            full((128, 1)),
            full((_EMBED, 1)),
            full((1,)),
        ],
        out_specs=pl.BlockSpec((_BLK,), lambda i: (i,)),
        out_shape=jax.ShapeDtypeStruct((_CB,), jnp.float32),
        scratch_shapes=[pltpu.VMEM((2 * _EMBED, _BLK), jnp.bfloat16)],
    )(u, b, w0, b0, w1, b1, w2, b2, w3, b3)


@jax.jit
def kernel(user_ids, book_ids, user_table, book_table,
           W0, b0, W1, b1, W2, b2, W3, b3):
    uids = user_ids.astype(jnp.int32)
    bids = book_ids.astype(jnp.int32)
    biases = (b0.reshape(-1, 1), b1.reshape(-1, 1), b2.reshape(-1, 1))
    gathered = [
        _sc_gather(user_table, book_table,
                   uids[c * _CB:(c + 1) * _CB], bids[c * _CB:(c + 1) * _CB])
        for c in range(_CHUNKS)
    ]
    outs = [
        _mlp(u, b, W0, biases[0], W1, biases[1], W2, biases[2], W3, b3)
        for (u, b) in gathered
    ]
    return jnp.concatenate(outs)


# speedup vs baseline: 1.0662x; 1.0662x over previous
import functools

import jax
import jax.numpy as jnp
from jax import lax
from jax.experimental import pallas as pl
from jax.experimental.pallas import tpu as pltpu
from jax.experimental.pallas import tpu_sc as plsc

_EMBED = 128
_BATCH = 16384
_NC = 2
_NS = 16
_NW = _NC * _NS
_CHUNKS = 1
_CB = _BATCH // _CHUNKS
_B_PER_W = _CB // _NW


def _gather_body(user_tab, book_tab, uids, bids, u_out, b_out, idx_v, rows_v, sem):
    wid = lax.axis_index("s") * _NC + lax.axis_index("c")
    base = wid * _B_PER_W
    pltpu.sync_copy(uids.at[pl.ds(base, _B_PER_W)], idx_v)
    pltpu.async_copy(user_tab.at[idx_v], rows_v, sem).wait()
    pltpu.sync_copy(rows_v, u_out.at[pl.ds(base, _B_PER_W)])
    pltpu.sync_copy(bids.at[pl.ds(base, _B_PER_W)], idx_v)
    pltpu.async_copy(book_tab.at[idx_v], rows_v, sem).wait()
    pltpu.sync_copy(rows_v, b_out.at[pl.ds(base, _B_PER_W)])


_sc_gather = functools.partial(
    pl.kernel,
    mesh=plsc.VectorSubcoreMesh(core_axis_name="c", subcore_axis_name="s"),
    out_type=[
        jax.ShapeDtypeStruct((_CB, _EMBED), jnp.float32),
        jax.ShapeDtypeStruct((_CB, _EMBED), jnp.float32),
    ],
    scratch_types=[
        pltpu.VMEM((_B_PER_W,), jnp.int32),
        pltpu.VMEM((_B_PER_W, _EMBED), jnp.float32),
        pltpu.SemaphoreType.DMA,
    ],
)(_gather_body)


_BLK = 4096

_DNUMS = (((0,), (0,)), ((), ()))


def _mlp_body(u_ref, b_ref, w0_ref, b0_ref, w1_ref, b1_ref,
              w2_ref, b2_ref, w3_ref, b3_ref, out_ref, xt_ref):
    f32, bf16 = jnp.float32, jnp.bfloat16
    dot = functools.partial(lax.dot_general, dimension_numbers=_DNUMS,
                            preferred_element_type=f32)
    xt_ref[:_EMBED, :] = u_ref[...].astype(bf16).T
    xt_ref[_EMBED:, :] = b_ref[...].astype(bf16).T
    h = dot(w0_ref[...].astype(bf16), xt_ref[...])
    h = jnp.maximum(h + b0_ref[...], 0.0).astype(bf16)
    h = dot(w1_ref[...].astype(bf16), h)
    h = jnp.maximum(h + b1_ref[...], 0.0).astype(bf16)
    h = dot(w2_ref[...].astype(bf16), h)
    h = jnp.maximum(h + b2_ref[...], 0.0).astype(bf16)
    y = dot(w3_ref[...].astype(bf16), h)
    out_ref[...] = y[0] + b3_ref[0]


def _mlp(u, b, w0, b0, w1, b1, w2, b2, w3, b3):
    grid = _CB // _BLK
    full = lambda shape: pl.BlockSpec(shape, lambda i: (0,) * len(shape))
    return pl.pallas_call(
        _mlp_body,
        grid=(grid,),
        in_specs=[
            pl.BlockSpec((_BLK, _EMBED), lambda i: (i, 0)),
            pl.BlockSpec((_BLK, _EMBED), lambda i: (i, 0)),
            full((2 * _EMBED, 512)),
            full((512, 1)),
            full((512, 256)),
            full((256, 1)),
            full((256, 128)),
            full((128, 1)),
            full((_EMBED, 1)),
            full((1,)),
        ],
        out_specs=pl.BlockSpec((_BLK,), lambda i: (i,)),
        out_shape=jax.ShapeDtypeStruct((_CB,), jnp.float32),
        scratch_shapes=[pltpu.VMEM((2 * _EMBED, _BLK), jnp.bfloat16)],
    )(u, b, w0, b0, w1, b1, w2, b2, w3, b3)


@jax.jit
def kernel(user_ids, book_ids, user_table, book_table,
           W0, b0, W1, b1, W2, b2, W3, b3):
    u, b = _sc_gather(user_table, book_table,
                      user_ids.astype(jnp.int32), book_ids.astype(jnp.int32))
    return _mlp(u, b, W0, b0.reshape(-1, 1), W1, b1.reshape(-1, 1),
                W2, b2.reshape(-1, 1), W3, b3)
